# baseline (speedup 1.0000x reference)
import jax
import jax.numpy as jnp
from jax import lax
from jax.experimental import pallas as pl
from jax.experimental.pallas import tpu as pltpu
from jax.experimental.pallas import tpu_sc as plsc

D = 32
B = 16384
NC, NS, L = 2, 16, 16
NW = NC * NS
BPW = B // NW
GROUPS = BPW // L
GPR = 128 // D
OPW = BPW // GPR


def _rsqrt_newton(s):
    i = plsc.bitcast(s, jnp.int32)
    i = jnp.int32(0x5F3759DF) - lax.shift_right_logical(i, 1)
    y = plsc.bitcast(i, jnp.float32)
    half_s = 0.5 * s
    for _ in range(3):
        y = y * (1.5 - half_s * y * y)
    return y


def _normalize_rows(idx_v, land_v, rows_v):
    lanes = lax.iota(jnp.int32, L)

    def group_body(g, _):
        base = g * L
        row_ids = base + lanes
        sub = lax.bitwise_and(idx_v[pl.ds(base, L)], jnp.int32(GPR - 1))
        colbase = sub * D
        cols = [plsc.load_gather(land_v, [row_ids, colbase + j])
                for j in range(D)]
        s = cols[0] * cols[0]
        for j in range(1, D):
            s = s + cols[j] * cols[j]
        y = _rsqrt_newton(jnp.maximum(s, 1e-24))
        orow = lax.shift_right_logical(row_ids, 2)
        ocolbase = lax.shift_left(lax.bitwise_and(row_ids, jnp.int32(GPR - 1)),
                                  5)
        for j in range(D):
            plsc.store_scatter(rows_v, [orow, ocolbase + j], cols[j] * y)
        return _

    lax.fori_loop(0, GROUPS, group_body, None)


def _sc_body(ent_hbm, rel_hbm, ph, pr, pt, nh, nr, nt,
             o0, o1, o2, o3, o4, o5, idx_v, grp_v, land_v, rows_v, sem):
    wid = lax.axis_index("s") * NC + lax.axis_index("c")
    base = wid * BPW
    jobs = ((ent_hbm, ph, o0), (rel_hbm, pr, o1), (ent_hbm, pt, o2),
            (ent_hbm, nh, o3), (rel_hbm, nr, o4), (ent_hbm, nt, o5))

    def shift_body(i, _):
        grp_v[pl.ds(i * L, L)] = lax.shift_right_logical(
            idx_v[pl.ds(i * L, L)], 2)
        return _

    for table, idx_hbm, out_hbm in jobs:
        pltpu.sync_copy(idx_hbm.at[pl.ds(base, BPW)], idx_v)
        lax.fori_loop(0, BPW // L, shift_body, None)
        pltpu.async_copy(table.at[grp_v], land_v, sem).wait()
        _normalize_rows(idx_v, land_v, rows_v)
        pltpu.sync_copy(rows_v, out_hbm.at[pl.ds(wid * OPW, OPW)])


@jax.jit
def kernel(rv_ent_embeds, rel_embeds, rel_pos_hs, rel_pos_rs, rel_pos_ts,
           rel_neg_hs, rel_neg_rs, rel_neg_ts):
    ent4 = rv_ent_embeds.reshape(-1, 128)
    rel4 = rel_embeds.reshape(-1, 128)
    out = jax.ShapeDtypeStruct((B // GPR, 128), jnp.float32)
    mesh = plsc.VectorSubcoreMesh(core_axis_name="c", subcore_axis_name="s",
                                  num_cores=NC, num_subcores=NS)
    run = pl.kernel(
        _sc_body,
        out_type=(out,) * 6,
        mesh=mesh,
        compiler_params=pltpu.CompilerParams(needs_layout_passes=False),
        scratch_types=[
            pltpu.VMEM((BPW,), jnp.int32),
            pltpu.VMEM((BPW,), jnp.int32),
            pltpu.VMEM((BPW, 128), jnp.float32),
            pltpu.VMEM((OPW, 128), jnp.float32),
            pltpu.SemaphoreType.DMA,
        ],
    )
    outs = run(ent4, rel4, rel_pos_hs, rel_pos_rs,
               rel_pos_ts, rel_neg_hs, rel_neg_rs, rel_neg_ts)
    return tuple(o.reshape(B, D) for o in outs)


# speedup vs baseline: 1.4872x; 1.4872x over previous
import jax
import jax.numpy as jnp
from jax import lax
from jax.experimental import pallas as pl
from jax.experimental.pallas import tpu as pltpu
from jax.experimental.pallas import tpu_sc as plsc

D = 32
B = 16384
NC, NS, L = 2, 16, 16
NW = NC * NS
BPW = B // NW
GROUPS = BPW // L


def _rsqrt_newton(s):
    i = plsc.bitcast(s, jnp.int32)
    i = jnp.int32(0x5F3759DF) - lax.shift_right_logical(i, 1)
    y = plsc.bitcast(i, jnp.float32)
    half_s = 0.5 * s
    for _ in range(3):
        y = y * (1.5 - half_s * y * y)
    return y


def _normalize_rows(rows_v):
    lanes = lax.iota(jnp.int32, L)
    col_ids = [jnp.full((L,), j, dtype=jnp.int32) for j in range(D)]

    def group_body(g, _):
        row_ids = g * L + lanes
        cols = [plsc.load_gather(rows_v, [row_ids, col_ids[j]])
                for j in range(D)]
        s = cols[0] * cols[0]
        for j in range(1, D):
            s = s + cols[j] * cols[j]
        y = _rsqrt_newton(jnp.maximum(s, 1e-24))
        for j in range(D):
            plsc.store_scatter(rows_v, [row_ids, col_ids[j]], cols[j] * y)
        return _

    lax.fori_loop(0, GROUPS, group_body, None)


def _sc_body(ent_hbm, rel_hbm, ph, pr, pt, nh, nr, nt,
             o0, o1, o2, o3, o4, o5, idx_v, rows_v, sem):
    wid = lax.axis_index("s") * NC + lax.axis_index("c")
    base = wid * BPW
    jobs = ((ent_hbm, ph, o0), (rel_hbm, pr, o1), (ent_hbm, pt, o2),
            (ent_hbm, nh, o3), (rel_hbm, nr, o4), (ent_hbm, nt, o5))

    for table, idx_hbm, out_hbm in jobs:
        pltpu.sync_copy(idx_hbm.at[pl.ds(base, BPW)], idx_v)

        def row_body(g, _):
            chunk = idx_v[pl.ds(g * L, L)]
            for j in range(L):
                pltpu.async_copy(table.at[pl.ds(chunk[j], 1)],
                                 rows_v.at[pl.ds(g * L + j, 1)], sem)
            return _

        lax.fori_loop(0, GROUPS, row_body, None)
        pltpu.make_async_copy(table.at[pl.ds(0, BPW)], rows_v, sem).wait()
        _normalize_rows(rows_v)
        pltpu.sync_copy(rows_v, out_hbm.at[pl.ds(base, BPW)])


@jax.jit
def kernel(rv_ent_embeds, rel_embeds, rel_pos_hs, rel_pos_rs, rel_pos_ts,
           rel_neg_hs, rel_neg_rs, rel_neg_ts):
    out = jax.ShapeDtypeStruct((B, D), jnp.float32)
    mesh = plsc.VectorSubcoreMesh(core_axis_name="c", subcore_axis_name="s",
                                  num_cores=NC, num_subcores=NS)
    run = pl.kernel(
        _sc_body,
        out_type=(out,) * 6,
        mesh=mesh,
        compiler_params=pltpu.CompilerParams(needs_layout_passes=False,
                                             use_tc_tiling_on_sc=True),
        scratch_types=[
            pltpu.VMEM((BPW,), jnp.int32),
            pltpu.VMEM((BPW, D), jnp.float32),
            pltpu.SemaphoreType.DMA,
        ],
    )
    return run(rv_ent_embeds, rel_embeds, rel_pos_hs, rel_pos_rs,
               rel_pos_ts, rel_neg_hs, rel_neg_rs, rel_neg_ts)
